# baseline (speedup 1.0000x reference)
import functools

import jax
import jax.numpy as jnp
from jax import lax
from jax.experimental import pallas as pl
from jax.experimental.pallas import tpu as pltpu
from jax.experimental.pallas import tpu_sc as plsc


@functools.lru_cache(maxsize=None)
def _make_gather(V, D, B):
    info = plsc.get_sparse_core_info()
    NC, NS = info.num_cores, info.num_subcores
    NW = NC * NS
    assert D == 64 and B % NW == 0
    b_per_w = B // NW
    SLOTS = 64
    mesh = plsc.VectorSubcoreMesh(core_axis_name="c", subcore_axis_name="s")

    @functools.partial(
        pl.kernel,
        mesh=mesh,
        out_type=jax.ShapeDtypeStruct((8, 8, B), jnp.float32),
        scratch_types=[
            pltpu.VMEM((b_per_w,), jnp.int32),
            pltpu.VMEM((8, 8, SLOTS * 16), jnp.float32),
            pltpu.VMEM((8, 8, b_per_w), jnp.float32),
            pltpu.SemaphoreType.DMA,
        ],
        compiler_params=pltpu.CompilerParams(
            needs_layout_passes=False,
            disable_bounds_checks=True,
            disable_semaphore_checks=True,
        ),
    )
    def k(t3_hbm, idx_hbm, outT_hbm, idx_v, ring_v, outT_v, sem1):
        wid = lax.axis_index("s") * NC + lax.axis_index("c")
        base = wid * b_per_w
        pltpu.sync_copy(idx_hbm.at[pl.ds(base, b_per_w)], idx_v)
        lanes = lax.iota(jnp.int32, 16)
        n_groups = b_per_w // 16
        lanes16 = lanes * 16

        def fire16(g):
            v = idx_v[pl.ds(g * 16, 16)]
            gslot = (g & (SLOTS // 16 - 1)) * 16
            for j in range(16):
                sb = jnp.take(v, jnp.full((16,), j, jnp.int32))
                a = lax.shift_right_logical(jnp.max(sb, axis=0), 4) * 16
                pltpu.make_async_copy(
                    t3_hbm.at[:, :, pl.ds(a, 16)],
                    ring_v.at[:, :, pl.ds((gslot + j) * 16, 16)],
                    sem1,
                ).start()

        def drain16():
            pltpu.make_async_copy(
                t3_hbm.at[:, :, pl.ds(0, 256)],
                ring_v.at[:, :, pl.ds(0, 256)],
                sem1,
            ).wait()

        def select16(g):
            v = idx_v[pl.ds(g * 16, 16)]
            pos = ((g & (SLOTS // 16 - 1)) * 256 + lanes16) + (v & 15)
            outv = g * 16 + lanes
            for c in range(D):
                cbv = jnp.full((16,), c >> 3, jnp.int32)
                civ = jnp.full((16,), c & 7, jnp.int32)
                vals = plsc.load_gather(ring_v, [cbv, civ, pos])
                plsc.store_scatter(outT_v, [cbv, civ, outv], vals)

        lax.fori_loop(0, 3, lambda g, c: (fire16(g), c)[1], 0)

        def body(g, carry):
            @pl.when(g + 3 < n_groups)
            def _():
                fire16(g + 3)

            drain16()
            select16(g)
            return carry

        lax.fori_loop(0, n_groups, body, 0)
        pltpu.sync_copy(outT_v, outT_hbm.at[:, :, pl.ds(base, b_per_w)])

    return k


def kernel(data, indices):
    V, D = data.shape
    B = indices.shape[0]
    idx = indices.reshape(B).astype(jnp.int32)
    t3 = data.T.reshape(8, 8, V)
    outT = _make_gather(V, D, B)(t3, idx)
    return outT.reshape(D, B).T


# speedup vs baseline: 1.0039x; 1.0039x over previous
import functools

import jax
import jax.numpy as jnp
from jax import lax
from jax.experimental import pallas as pl
from jax.experimental.pallas import tpu as pltpu
from jax.experimental.pallas import tpu_sc as plsc


@functools.lru_cache(maxsize=None)
def _make_gather(V, D, B):
    info = plsc.get_sparse_core_info()
    NC, NS = info.num_cores, info.num_subcores
    NW = NC * NS
    assert D == 64 and B % NW == 0
    b_per_w = B // NW
    SLOTS = 64
    mesh = plsc.VectorSubcoreMesh(core_axis_name="c", subcore_axis_name="s")

    @functools.partial(
        pl.kernel,
        mesh=mesh,
        out_type=jax.ShapeDtypeStruct((8, 8, B), jnp.float32),
        scratch_types=[
            pltpu.VMEM((b_per_w,), jnp.int32),
            pltpu.VMEM((8, 8, SLOTS * 16), jnp.float32),
            pltpu.VMEM((8, 8, b_per_w), jnp.float32),
            pltpu.SemaphoreType.DMA,
        ],
        compiler_params=pltpu.CompilerParams(needs_layout_passes=False),
    )
    def k(t3_hbm, idx_hbm, outT_hbm, idx_v, ring_v, outT_v, sem1):
        wid = lax.axis_index("s") * NC + lax.axis_index("c")
        base = wid * b_per_w
        pltpu.sync_copy(idx_hbm.at[pl.ds(base, b_per_w)], idx_v)
        lanes = lax.iota(jnp.int32, 16)
        n_groups = b_per_w // 16
        lanes16 = lanes * 16

        def fire16(g):
            v = idx_v[pl.ds(g * 16, 16)]
            gslot = (g & (SLOTS // 16 - 1)) * 16
            for j in range(16):
                sb = jnp.take(v, jnp.full((16,), j, jnp.int32))
                a = lax.shift_right_logical(jnp.max(sb, axis=0), 4) * 16
                pltpu.make_async_copy(
                    t3_hbm.at[:, :, pl.ds(a, 16)],
                    ring_v.at[:, :, pl.ds((gslot + j) * 16, 16)],
                    sem1,
                ).start()

        def drain16():
            pltpu.make_async_copy(
                t3_hbm.at[:, :, pl.ds(0, 256)],
                ring_v.at[:, :, pl.ds(0, 256)],
                sem1,
            ).wait()

        def select16(g):
            v = idx_v[pl.ds(g * 16, 16)]
            pos = ((g & (SLOTS // 16 - 1)) * 256 + lanes16) + (v & 15)
            outv = g * 16 + lanes
            for c in range(D):
                cbv = jnp.full((16,), c >> 3, jnp.int32)
                civ = jnp.full((16,), c & 7, jnp.int32)
                vals = plsc.load_gather(ring_v, [cbv, civ, pos])
                plsc.store_scatter(outT_v, [cbv, civ, outv], vals)

        lax.fori_loop(0, 3, lambda g, c: (fire16(g), c)[1], 0)

        def body(g, carry):
            @pl.when(g + 3 < n_groups)
            def _():
                fire16(g + 3)

            drain16()
            select16(g)
            return carry

        lax.fori_loop(0, n_groups, body, 0)
        pltpu.sync_copy(outT_v, outT_hbm.at[:, :, pl.ds(base, b_per_w)])

    return k


def kernel(data, indices):
    V, D = data.shape
    B = indices.shape[0]
    idx = indices.reshape(B).astype(jnp.int32)
    t3 = data.T.reshape(8, 8, V)
    outT = _make_gather(V, D, B)(t3, idx)
    return outT.reshape(D, B).T
